# baseline (speedup 1.0000x reference)
import functools

import jax
import jax.numpy as jnp
from jax import lax
from jax.experimental import pallas as pl
from jax.experimental.pallas import tpu as pltpu
from jax.experimental.pallas import tpu_sc as plsc

N_TOKENS = 16384
N_CODES = 8192
DIM = 256
COMMIT = 0.25

HALF = N_TOKENS // 2
M_BLK = 1024
H_BLOCKS = HALF // M_BLK

L_BLK = 1024
L_BLOCKS = HALF // L_BLK

NC, NS = 2, 16
NW = NC * NS
ROWS_PER_W = HALF // NW
CHUNK = 128
N_CHUNKS = ROWS_PER_W // CHUNK


def _argmax_scores(x_ref, cbn, idx_ref):
    x = x_ref[...]
    nrm = jnp.clip(jnp.sqrt(jnp.sum(x * x, axis=1, keepdims=True)),
                   1e-8, None)
    xn = x / nrm
    s = lax.dot_general(xn, cbn, (((1,), (1,)), ((), ())),
                        preferred_element_type=jnp.float32)
    idx_ref[0, 0, :] = jnp.argmax(s, axis=1).astype(jnp.int32)


def _argmax_first_body(x_ref, cb_ref, idx_ref, cbu_ref, cbn_ref):
    @pl.when(pl.program_id(0) == 0)
    def _():
        cb = cb_ref[...]
        cn = jnp.clip(jnp.sqrt(jnp.sum(cb * cb, axis=1, keepdims=True)),
                      1e-8, None)
        cbn_ref[...] = cb / cn
        cbu_ref[...] = cb.reshape(N_CODES // 4, 8, 128)

    _argmax_scores(x_ref, cbn_ref[...], idx_ref)


def _argmax_second_body(x_ref, cbn_ref, idx_ref):
    _argmax_scores(x_ref, cbn_ref[...], idx_ref)


def _argmax_first(x, cb):
    return pl.pallas_call(
        _argmax_first_body,
        grid=(H_BLOCKS,),
        in_specs=[
            pl.BlockSpec((M_BLK, DIM), lambda i: (i, 0)),
            pl.BlockSpec((N_CODES, DIM), lambda i: (0, 0)),
        ],
        out_specs=[
            pl.BlockSpec((1, 1, M_BLK), lambda i: (i, 0, 0)),
            pl.BlockSpec((N_CODES // 4, 8, 128), lambda i: (0, 0, 0)),
            pl.BlockSpec((N_CODES, DIM), lambda i: (0, 0)),
        ],
        out_shape=[
            jax.ShapeDtypeStruct((H_BLOCKS, 1, M_BLK), jnp.int32),
            jax.ShapeDtypeStruct((N_CODES // 4, 8, 128), jnp.float32),
            jax.ShapeDtypeStruct((N_CODES, DIM), jnp.float32),
        ],
        compiler_params=pltpu.CompilerParams(vmem_limit_bytes=110 * 2**20),
    )(x, cb)


def _argmax_second(x, cbn):
    return pl.pallas_call(
        _argmax_second_body,
        grid=(H_BLOCKS,),
        in_specs=[
            pl.BlockSpec((M_BLK, DIM), lambda i: (i + H_BLOCKS, 0)),
            pl.BlockSpec((N_CODES, DIM), lambda i: (0, 0)),
        ],
        out_specs=pl.BlockSpec((1, 1, M_BLK), lambda i: (i, 0, 0)),
        out_shape=jax.ShapeDtypeStruct((H_BLOCKS, 1, M_BLK), jnp.int32),
        compiler_params=pltpu.CompilerParams(vmem_limit_bytes=110 * 2**20),
    )(x, cbn)


def _loss_first_body(x_ref, qv_ref, q_ref, acc_ref):
    q = qv_ref[...].reshape(L_BLK, DIM)
    q_ref[...] = q
    d = q - x_ref[...]

    @pl.when(pl.program_id(0) == 0)
    def _():
        acc_ref[0, 0] = 0.0

    acc_ref[0, 0] += jnp.sum(d * d)


def _loss_second_body(x_ref, qv_ref, qin_ref, q_ref, acc_ref):
    del qin_ref
    _loss_first_body(x_ref, qv_ref, q_ref, acc_ref)


def _loss_half(x, qv, half, qprev=None):
    off = half * L_BLOCKS
    in_specs = [
        pl.BlockSpec((L_BLK, DIM), lambda i: (i + off, 0)),
        pl.BlockSpec((L_BLK // 4, 8, 128), lambda i: (i, 0, 0)),
    ]
    args = [x, qv]
    body = _loss_first_body
    aliases = {}
    if qprev is not None:
        in_specs.append(pl.BlockSpec(memory_space=pl.ANY))
        args.append(qprev)
        body = _loss_second_body
        aliases = {2: 0}
    return pl.pallas_call(
        body,
        grid=(L_BLOCKS,),
        in_specs=in_specs,
        out_specs=[
            pl.BlockSpec((L_BLK, DIM), lambda i: (i + off, 0)),
            pl.BlockSpec((1, 1), lambda i: (0, 0), memory_space=pltpu.SMEM),
        ],
        out_shape=[
            jax.ShapeDtypeStruct((N_TOKENS, DIM), jnp.float32),
            jax.ShapeDtypeStruct((1, 1), jnp.float32),
        ],
        input_output_aliases=aliases,
    )(*args)


def _gather_body(cb_hbm, idx_hbm, out_hbm, idx_v, rows_v, sem):
    wid = lax.axis_index("s") * NC + lax.axis_index("c")
    pltpu.sync_copy(idx_hbm.at[pl.ds(wid * N_CHUNKS, N_CHUNKS)], idx_v)
    for c in range(N_CHUNKS):
        pltpu.async_copy(cb_hbm.at[idx_v.at[c]], rows_v, sem).wait()
        pltpu.sync_copy(
            rows_v, out_hbm.at[pl.ds(wid * ROWS_PER_W + c * CHUNK, CHUNK)])


@functools.lru_cache(maxsize=None)
def _sc_gather():
    return pl.kernel(
        _gather_body,
        out_type=jax.ShapeDtypeStruct((HALF, DIM), jnp.float32),
        mesh=plsc.VectorSubcoreMesh(core_axis_name="c", subcore_axis_name="s",
                                    num_cores=NC, num_subcores=NS),
        scratch_types=[
            pltpu.VMEM((N_CHUNKS, CHUNK), jnp.int32),
            pltpu.VMEM((CHUNK, DIM), jnp.float32),
            pltpu.SemaphoreType.DMA,
        ],
        compiler_params=pltpu.CompilerParams(use_tc_tiling_on_sc=False),
    )


def kernel(inputs, codebook):
    idx3a, cbu3, cbn = _argmax_first(inputs, codebook)
    idx3b = _argmax_second(inputs, cbn)

    cbu = cbu3.reshape(N_CODES, DIM)
    g0 = _sc_gather()(cbu, idx3a.reshape(NW * N_CHUNKS, CHUNK))
    g1 = _sc_gather()(cbu, idx3b.reshape(NW * N_CHUNKS, CHUNK))

    qv0 = g0.reshape(HALF // 4, 8, 128)
    qv1 = g1.reshape(HALF // 4, 8, 128)

    qhalf, p0 = _loss_half(inputs, qv0, 0)
    quantized, p1 = _loss_half(inputs, qv1, 1, qprev=qhalf)

    m = (p0[0, 0] + p1[0, 0]) / (N_TOKENS * DIM)
    loss = m + COMMIT * m
    return quantized, loss


# speedup vs baseline: 1.0222x; 1.0222x over previous
import functools

import jax
import jax.numpy as jnp
from jax import lax
from jax.experimental import pallas as pl
from jax.experimental.pallas import tpu as pltpu
from jax.experimental.pallas import tpu_sc as plsc

N_TOKENS = 16384
N_CODES = 8192
DIM = 256
COMMIT = 0.25

HALF = N_TOKENS // 2
M_BLK = 1024
H_BLOCKS = HALF // M_BLK

L_BLK = 2048
L_BLOCKS = HALF // L_BLK

NC, NS = 2, 16
NW = NC * NS
ROWS_PER_W = HALF // NW
CHUNK = 128
N_CHUNKS = ROWS_PER_W // CHUNK


def _argmax_scores(x_ref, cbn, idx_ref):
    x = x_ref[...]
    nrm = jnp.clip(jnp.sqrt(jnp.sum(x * x, axis=1, keepdims=True)),
                   1e-8, None)
    xn = x / nrm
    s = lax.dot_general(xn, cbn, (((1,), (1,)), ((), ())),
                        preferred_element_type=jnp.float32)
    idx_ref[0, 0, :] = jnp.argmax(s, axis=1).astype(jnp.int32)


def _argmax_first_body(x_ref, cb_ref, idx_ref, cbu_ref, cbn_ref):
    @pl.when(pl.program_id(0) == 0)
    def _():
        cb = cb_ref[...]
        cn = jnp.clip(jnp.sqrt(jnp.sum(cb * cb, axis=1, keepdims=True)),
                      1e-8, None)
        cbn_ref[...] = cb / cn
        cbu_ref[...] = cb.reshape(N_CODES // 4, 8, 128)

    _argmax_scores(x_ref, cbn_ref[...], idx_ref)


def _argmax_second_body(x_ref, cbn_ref, idx_ref):
    _argmax_scores(x_ref, cbn_ref[...], idx_ref)


def _argmax_first(x, cb):
    return pl.pallas_call(
        _argmax_first_body,
        grid=(H_BLOCKS,),
        in_specs=[
            pl.BlockSpec((M_BLK, DIM), lambda i: (i, 0)),
            pl.BlockSpec((N_CODES, DIM), lambda i: (0, 0)),
        ],
        out_specs=[
            pl.BlockSpec((1, 1, M_BLK), lambda i: (i, 0, 0)),
            pl.BlockSpec((N_CODES // 4, 8, 128), lambda i: (0, 0, 0)),
            pl.BlockSpec((N_CODES, DIM), lambda i: (0, 0)),
        ],
        out_shape=[
            jax.ShapeDtypeStruct((H_BLOCKS, 1, M_BLK), jnp.int32),
            jax.ShapeDtypeStruct((N_CODES // 4, 8, 128), jnp.float32),
            jax.ShapeDtypeStruct((N_CODES, DIM), jnp.float32),
        ],
        compiler_params=pltpu.CompilerParams(vmem_limit_bytes=110 * 2**20),
    )(x, cb)


def _argmax_second(x, cbn):
    return pl.pallas_call(
        _argmax_second_body,
        grid=(H_BLOCKS,),
        in_specs=[
            pl.BlockSpec((M_BLK, DIM), lambda i: (i + H_BLOCKS, 0)),
            pl.BlockSpec((N_CODES, DIM), lambda i: (0, 0)),
        ],
        out_specs=pl.BlockSpec((1, 1, M_BLK), lambda i: (i, 0, 0)),
        out_shape=jax.ShapeDtypeStruct((H_BLOCKS, 1, M_BLK), jnp.int32),
        compiler_params=pltpu.CompilerParams(vmem_limit_bytes=110 * 2**20),
    )(x, cbn)


def _loss_first_body(x_ref, qv_ref, q_ref, acc_ref):
    q = qv_ref[...].reshape(L_BLK, DIM)
    q_ref[...] = q
    d = q - x_ref[...]

    @pl.when(pl.program_id(0) == 0)
    def _():
        acc_ref[0, 0] = 0.0

    acc_ref[0, 0] += jnp.sum(d * d)


def _loss_second_body(x_ref, qv_ref, qin_ref, q_ref, acc_ref):
    del qin_ref
    _loss_first_body(x_ref, qv_ref, q_ref, acc_ref)


def _loss_half(x, qv, half, qprev=None):
    off = half * L_BLOCKS
    in_specs = [
        pl.BlockSpec((L_BLK, DIM), lambda i: (i + off, 0)),
        pl.BlockSpec((L_BLK // 4, 8, 128), lambda i: (i, 0, 0)),
    ]
    args = [x, qv]
    body = _loss_first_body
    aliases = {}
    if qprev is not None:
        in_specs.append(pl.BlockSpec(memory_space=pl.ANY))
        args.append(qprev)
        body = _loss_second_body
        aliases = {2: 0}
    return pl.pallas_call(
        body,
        grid=(L_BLOCKS,),
        in_specs=in_specs,
        out_specs=[
            pl.BlockSpec((L_BLK, DIM), lambda i: (i + off, 0)),
            pl.BlockSpec((1, 1), lambda i: (0, 0), memory_space=pltpu.SMEM),
        ],
        out_shape=[
            jax.ShapeDtypeStruct((N_TOKENS, DIM), jnp.float32),
            jax.ShapeDtypeStruct((1, 1), jnp.float32),
        ],
        input_output_aliases=aliases,
    )(*args)


def _gather_body(cb_hbm, idx_hbm, out_hbm, idx_v, rows_v, sem):
    wid = lax.axis_index("s") * NC + lax.axis_index("c")
    pltpu.sync_copy(idx_hbm.at[pl.ds(wid * N_CHUNKS, N_CHUNKS)], idx_v)
    for c in range(N_CHUNKS):
        pltpu.async_copy(cb_hbm.at[idx_v.at[c]], rows_v, sem).wait()
        pltpu.sync_copy(
            rows_v, out_hbm.at[pl.ds(wid * ROWS_PER_W + c * CHUNK, CHUNK)])


@functools.lru_cache(maxsize=None)
def _sc_gather():
    return pl.kernel(
        _gather_body,
        out_type=jax.ShapeDtypeStruct((HALF, DIM), jnp.float32),
        mesh=plsc.VectorSubcoreMesh(core_axis_name="c", subcore_axis_name="s",
                                    num_cores=NC, num_subcores=NS),
        scratch_types=[
            pltpu.VMEM((N_CHUNKS, CHUNK), jnp.int32),
            pltpu.VMEM((CHUNK, DIM), jnp.float32),
            pltpu.SemaphoreType.DMA,
        ],
        compiler_params=pltpu.CompilerParams(use_tc_tiling_on_sc=False),
    )


def kernel(inputs, codebook):
    idx3a, cbu3, cbn = _argmax_first(inputs, codebook)
    idx3b = _argmax_second(inputs, cbn)

    cbu = cbu3.reshape(N_CODES, DIM)
    g0 = _sc_gather()(cbu, idx3a.reshape(NW * N_CHUNKS, CHUNK))
    g1 = _sc_gather()(cbu, idx3b.reshape(NW * N_CHUNKS, CHUNK))

    qv0 = g0.reshape(HALF // 4, 8, 128)
    qv1 = g1.reshape(HALF // 4, 8, 128)

    qhalf, p0 = _loss_half(inputs, qv0, 0)
    quantized, p1 = _loss_half(inputs, qv1, 1, qprev=qhalf)

    m = (p0[0, 0] + p1[0, 0]) / (N_TOKENS * DIM)
    loss = m + COMMIT * m
    return quantized, loss
